# Initial kernel scaffold; baseline (speedup 1.0000x reference)
import jax
import jax.numpy as jnp
from jax.experimental import pallas as pl


def kernel(x, edge_index, node_to_graph_map, W_embed, W_msg, Wz, Uz, bz, Wr, Ur, br, Wh, Uh, bh, W_ex, b_ex):
    raise NotImplementedError("write your pallas kernel here")


# speedup vs baseline: 17.3477x; 17.3477x over previous
import functools

import jax
import jax.numpy as jnp
from jax import lax
from jax.experimental import pallas as pl
from jax.experimental.pallas import tpu as pltpu
from jax.experimental.pallas import tpu_sc as plsc

N = 10000
E = 320000
D_IN = 128
H = 16
L = 4
G = 64

NW = 32
CHUNK = 128
NB = 8
CH = 80
E_PAD = NW * CH * CHUNK
N_PAD = 10112
RPT = N_PAD // 16
N_DUMMY = N

_f32 = jnp.float32


def _sc_agg(hm_pad, src3, dst3, zeros_tile):
    mesh = plsc.VectorSubcoreMesh(core_axis_name="c", subcore_axis_name="s")

    @functools.partial(
        pl.kernel,
        mesh=mesh,
        compiler_params=pltpu.CompilerParams(use_tc_tiling_on_sc=False),
        out_type=jax.ShapeDtypeStruct((2 * N_PAD, H), _f32),
        scratch_types=[
            pltpu.VMEM((NB, CHUNK), jnp.int32),
            pltpu.VMEM((NB, CHUNK), jnp.int32),
            pltpu.VMEM((CHUNK, H), _f32),
            pltpu.VMEM((RPT, H), _f32),
            pltpu.VMEM_SHARED((N_PAD, H), _f32),
            pltpu.VMEM_SHARED((N_PAD, H), _f32),
            pltpu.SemaphoreType.DMA,
        ],
    )
    def k(hm_hbm, src_hbm, dst_hbm, zeros_hbm, out_hbm,
          src_v, dst_v, rows_v, stage_v, hm_sh, agg_sh, sem):
        c = lax.axis_index("c")
        s = lax.axis_index("s")
        w = c * 16 + s
        r0 = s * RPT

        pltpu.sync_copy(zeros_hbm, stage_v)
        pltpu.sync_copy(stage_v, agg_sh.at[pl.ds(r0, RPT)])
        pltpu.sync_copy(hm_hbm.at[pl.ds(r0, RPT)], stage_v)
        pltpu.sync_copy(stage_v, hm_sh.at[pl.ds(r0, RPT)])
        plsc.subcore_barrier()

        def body(b, carry):
            pltpu.sync_copy(src_hbm.at[pl.ds(w * CH + b * NB, NB)], src_v)
            pltpu.sync_copy(dst_hbm.at[pl.ds(w * CH + b * NB, NB)], dst_v)
            for g in range(NB):
                pltpu.async_copy(hm_sh.at[src_v.at[g]], rows_v, sem).wait()
                pltpu.sync_copy(rows_v, agg_sh.at[dst_v.at[g]], add=True)
            return carry

        lax.fori_loop(0, CH // NB, body, 0)
        plsc.subcore_barrier()
        pltpu.sync_copy(agg_sh.at[pl.ds(r0, RPT)],
                        out_hbm.at[pl.ds(c * N_PAD + r0, RPT)])

    return k(hm_pad, src3, dst3, zeros_tile)


def _dot(a, b):
    return jnp.dot(a, b, preferred_element_type=_f32)


def _embed_body(x_ref, we_ref, wm_ref, h_ref, hm_ref):
    h = _dot(x_ref[...], we_ref[...])
    h_ref[...] = h
    hm_ref[...] = _dot(h, wm_ref[...])


def _embed_call(x_pad, W_embed, W_msg0):
    return pl.pallas_call(
        _embed_body,
        out_shape=(jax.ShapeDtypeStruct((N_PAD, H), _f32),
                   jax.ShapeDtypeStruct((N_PAD, H), _f32)),
    )(x_pad, W_embed, W_msg0)


def _gru(h, agg, wz, uz, bz, wr, ur, br, wh, uh, bh):
    z = jax.nn.sigmoid(_dot(agg, wz) + _dot(h, uz) + bz)
    r = jax.nn.sigmoid(_dot(agg, wr) + _dot(h, ur) + br)
    h_tilde = jnp.tanh(_dot(agg, wh) + _dot(r * h, uh) + bh)
    return (1.0 - z) * h + z * h_tilde


def _gru_body(h_ref, p_ref, wz_ref, uz_ref, bz_ref, wr_ref, ur_ref, br_ref,
              wh_ref, uh_ref, bh_ref, wm_ref, hout_ref, hm_ref):
    agg = p_ref[0] + p_ref[1]
    h = _gru(h_ref[...], agg, wz_ref[...], uz_ref[...], bz_ref[...],
             wr_ref[...], ur_ref[...], br_ref[...],
             wh_ref[...], uh_ref[...], bh_ref[...])
    hout_ref[...] = h
    hm_ref[...] = _dot(h, wm_ref[...])


def _gru_call(h, parts, wz, uz, bz, wr, ur, br, wh, uh, bh, wm):
    return pl.pallas_call(
        _gru_body,
        out_shape=(jax.ShapeDtypeStruct((N_PAD, H), _f32),
                   jax.ShapeDtypeStruct((N_PAD, H), _f32)),
    )(h, parts, wz, uz, bz, wr, ur, br, wh, uh, bh, wm)


def _gru_ex_body(h_ref, p_ref, wz_ref, uz_ref, bz_ref, wr_ref, ur_ref, br_ref,
                 wh_ref, uh_ref, bh_ref, mrow_ref, mcol_ref, wexa_ref,
                 wexb_ref, bex_ref, wm_ref, hout_ref, hm_ref):
    agg = p_ref[0] + p_ref[1]
    h = _gru(h_ref[...], agg, wz_ref[...], uz_ref[...], bz_ref[...],
             wr_ref[...], ur_ref[...], br_ref[...],
             wh_ref[...], uh_ref[...], bh_ref[...])
    ids_g = lax.broadcasted_iota(jnp.int32, (G, N_PAD), 0)
    oh_t = (ids_g == mrow_ref[...]).astype(_f32)
    sums = _dot(oh_t, h)
    cnt = jnp.sum(oh_t, axis=1, keepdims=True)
    mean = sums / jnp.maximum(cnt, 1.0)
    ids_n = lax.broadcasted_iota(jnp.int32, (N_PAD, G), 1)
    oh = (mcol_ref[...] == ids_n).astype(_f32)
    per_node = _dot(oh, mean)
    h = h + jnp.tanh(_dot(h, wexa_ref[...]) + _dot(per_node, wexb_ref[...])
                     + bex_ref[...])
    hout_ref[...] = h
    hm_ref[...] = _dot(h, wm_ref[...])


def _gru_ex_call(h, parts, wz, uz, bz, wr, ur, br, wh, uh, bh,
                 mrow, mcol, wexa, wexb, bex, wm):
    return pl.pallas_call(
        _gru_ex_body,
        out_shape=(jax.ShapeDtypeStruct((N_PAD, H), _f32),
                   jax.ShapeDtypeStruct((N_PAD, H), _f32)),
    )(h, parts, wz, uz, bz, wr, ur, br, wh, uh, bh,
      mrow, mcol, wexa, wexb, bex, wm)


def kernel(x, edge_index, node_to_graph_map, W_embed, W_msg,
           Wz, Uz, bz, Wr, Ur, br, Wh, Uh, bh, W_ex, b_ex):
    x_pad = jnp.pad(x, ((0, N_PAD - N), (0, 0)))
    mrow = jnp.pad(node_to_graph_map, (0, N_PAD - N),
                   constant_values=G).reshape(1, N_PAD)
    mcol = mrow.reshape(N_PAD, 1)
    src3 = jnp.pad(edge_index[0], (0, E_PAD - E)).reshape(NW * CH, CHUNK)
    dst3 = jnp.pad(edge_index[1], (0, E_PAD - E),
                   constant_values=N_DUMMY).reshape(NW * CH, CHUNK)
    zeros_tile = jnp.zeros((RPT, H), _f32)

    h, hm = _embed_call(x_pad, W_embed, W_msg[0])
    ex_i = 0
    for l in range(L):
        parts = _sc_agg(hm, src3, dst3, zeros_tile).reshape(2, N_PAD, H)
        wm_next = W_msg[(l + 1) % L]
        args = (h, parts, Wz[l], Uz[l], bz[l].reshape(1, H),
                Wr[l], Ur[l], br[l].reshape(1, H),
                Wh[l], Uh[l], bh[l].reshape(1, H))
        if (l + 1) % 2 == 0:
            h, hm = _gru_ex_call(*args, mrow, mcol,
                                 W_ex[ex_i, :H], W_ex[ex_i, H:],
                                 b_ex[ex_i].reshape(1, H), wm_next)
            ex_i += 1
        else:
            h, hm = _gru_call(*args, wm_next)
    return h[:N]


# speedup vs baseline: 19.7264x; 1.1371x over previous
import functools

import jax
import jax.numpy as jnp
from jax import lax
from jax.experimental import pallas as pl
from jax.experimental.pallas import tpu as pltpu
from jax.experimental.pallas import tpu_sc as plsc

N = 10000
E = 320000
D_IN = 128
H = 16
L = 4
G = 64

NW = 32
CHUNK = 128
NB = 8
CH = 80
E_PAD = NW * CH * CHUNK
N_PAD = 10112
RPT = N_PAD // 16
N_DUMMY = N

_f32 = jnp.float32


def _sc_agg(hm_pad, src3, dst3, zeros_tile):
    mesh = plsc.VectorSubcoreMesh(core_axis_name="c", subcore_axis_name="s")

    @functools.partial(
        pl.kernel,
        mesh=mesh,
        compiler_params=pltpu.CompilerParams(use_tc_tiling_on_sc=False),
        out_type=jax.ShapeDtypeStruct((2 * N_PAD, H), _f32),
        scratch_types=[
            pltpu.VMEM((NB, CHUNK), jnp.int32),
            pltpu.VMEM((NB, CHUNK), jnp.int32),
            pltpu.VMEM((NB, CHUNK, H), _f32),
            pltpu.VMEM((RPT, H), _f32),
            pltpu.VMEM_SHARED((N_PAD, H), _f32),
            pltpu.VMEM_SHARED((N_PAD, H), _f32),
            [pltpu.SemaphoreType.DMA] * NB,
        ],
    )
    def k(hm_hbm, src_hbm, dst_hbm, zeros_hbm, out_hbm,
          src_v, dst_v, rows_v, stage_v, hm_sh, agg_sh, sems):
        c = lax.axis_index("c")
        s = lax.axis_index("s")
        w = c * 16 + s
        r0 = s * RPT

        pltpu.sync_copy(zeros_hbm, stage_v)
        pltpu.sync_copy(stage_v, agg_sh.at[pl.ds(r0, RPT)])
        pltpu.sync_copy(hm_hbm.at[pl.ds(r0, RPT)], stage_v)
        pltpu.sync_copy(stage_v, hm_sh.at[pl.ds(r0, RPT)])
        plsc.subcore_barrier()

        def body(b, carry):
            pltpu.sync_copy(src_hbm.at[pl.ds(w * CH + b * NB, NB)], src_v)
            pltpu.sync_copy(dst_hbm.at[pl.ds(w * CH + b * NB, NB)], dst_v)
            gathers = [
                pltpu.async_copy(hm_sh.at[src_v.at[g]], rows_v.at[g], sems[g])
                for g in range(NB)
            ]
            scatters = []
            for g in range(NB):
                gathers[g].wait()
                scatters.append(
                    pltpu.async_copy(rows_v.at[g], agg_sh.at[dst_v.at[g]],
                                     sems[g], add=True))
            for sc in scatters:
                sc.wait()
            return carry

        lax.fori_loop(0, CH // NB, body, 0)
        plsc.subcore_barrier()
        pltpu.sync_copy(agg_sh.at[pl.ds(r0, RPT)],
                        out_hbm.at[pl.ds(c * N_PAD + r0, RPT)])

    return k(hm_pad, src3, dst3, zeros_tile)


def _dot(a, b):
    return jnp.dot(a, b, preferred_element_type=_f32)


def _embed_body(x_ref, we_ref, wm_ref, h_ref, hm_ref):
    h = _dot(x_ref[...], we_ref[...])
    h_ref[...] = h
    hm_ref[...] = _dot(h, wm_ref[...])


def _embed_call(x_pad, W_embed, W_msg0):
    return pl.pallas_call(
        _embed_body,
        out_shape=(jax.ShapeDtypeStruct((N_PAD, H), _f32),
                   jax.ShapeDtypeStruct((N_PAD, H), _f32)),
    )(x_pad, W_embed, W_msg0)


def _gru(h, agg, wz, uz, bz, wr, ur, br, wh, uh, bh):
    z = jax.nn.sigmoid(_dot(agg, wz) + _dot(h, uz) + bz)
    r = jax.nn.sigmoid(_dot(agg, wr) + _dot(h, ur) + br)
    h_tilde = jnp.tanh(_dot(agg, wh) + _dot(r * h, uh) + bh)
    return (1.0 - z) * h + z * h_tilde


def _gru_body(h_ref, p_ref, wz_ref, uz_ref, bz_ref, wr_ref, ur_ref, br_ref,
              wh_ref, uh_ref, bh_ref, wm_ref, hout_ref, hm_ref):
    agg = p_ref[0] + p_ref[1]
    h = _gru(h_ref[...], agg, wz_ref[...], uz_ref[...], bz_ref[...],
             wr_ref[...], ur_ref[...], br_ref[...],
             wh_ref[...], uh_ref[...], bh_ref[...])
    hout_ref[...] = h
    hm_ref[...] = _dot(h, wm_ref[...])


def _gru_call(h, parts, wz, uz, bz, wr, ur, br, wh, uh, bh, wm):
    return pl.pallas_call(
        _gru_body,
        out_shape=(jax.ShapeDtypeStruct((N_PAD, H), _f32),
                   jax.ShapeDtypeStruct((N_PAD, H), _f32)),
    )(h, parts, wz, uz, bz, wr, ur, br, wh, uh, bh, wm)


def _gru_ex_body(h_ref, p_ref, wz_ref, uz_ref, bz_ref, wr_ref, ur_ref, br_ref,
                 wh_ref, uh_ref, bh_ref, mrow_ref, mcol_ref, wexa_ref,
                 wexb_ref, bex_ref, wm_ref, hout_ref, hm_ref):
    agg = p_ref[0] + p_ref[1]
    h = _gru(h_ref[...], agg, wz_ref[...], uz_ref[...], bz_ref[...],
             wr_ref[...], ur_ref[...], br_ref[...],
             wh_ref[...], uh_ref[...], bh_ref[...])
    ids_g = lax.broadcasted_iota(jnp.int32, (G, N_PAD), 0)
    oh_t = (ids_g == mrow_ref[...]).astype(_f32)
    sums = _dot(oh_t, h)
    cnt = jnp.sum(oh_t, axis=1, keepdims=True)
    mean = sums / jnp.maximum(cnt, 1.0)
    ids_n = lax.broadcasted_iota(jnp.int32, (N_PAD, G), 1)
    oh = (mcol_ref[...] == ids_n).astype(_f32)
    per_node = _dot(oh, mean)
    h = h + jnp.tanh(_dot(h, wexa_ref[...]) + _dot(per_node, wexb_ref[...])
                     + bex_ref[...])
    hout_ref[...] = h
    hm_ref[...] = _dot(h, wm_ref[...])


def _gru_ex_call(h, parts, wz, uz, bz, wr, ur, br, wh, uh, bh,
                 mrow, mcol, wexa, wexb, bex, wm):
    return pl.pallas_call(
        _gru_ex_body,
        out_shape=(jax.ShapeDtypeStruct((N_PAD, H), _f32),
                   jax.ShapeDtypeStruct((N_PAD, H), _f32)),
    )(h, parts, wz, uz, bz, wr, ur, br, wh, uh, bh,
      mrow, mcol, wexa, wexb, bex, wm)


def kernel(x, edge_index, node_to_graph_map, W_embed, W_msg,
           Wz, Uz, bz, Wr, Ur, br, Wh, Uh, bh, W_ex, b_ex):
    x_pad = jnp.pad(x, ((0, N_PAD - N), (0, 0)))
    mrow = jnp.pad(node_to_graph_map, (0, N_PAD - N),
                   constant_values=G).reshape(1, N_PAD)
    mcol = mrow.reshape(N_PAD, 1)
    src3 = jnp.pad(edge_index[0], (0, E_PAD - E)).reshape(NW * CH, CHUNK)
    dst3 = jnp.pad(edge_index[1], (0, E_PAD - E),
                   constant_values=N_DUMMY).reshape(NW * CH, CHUNK)
    zeros_tile = jnp.zeros((RPT, H), _f32)

    h, hm = _embed_call(x_pad, W_embed, W_msg[0])
    ex_i = 0
    for l in range(L):
        parts = _sc_agg(hm, src3, dst3, zeros_tile).reshape(2, N_PAD, H)
        wm_next = W_msg[(l + 1) % L]
        args = (h, parts, Wz[l], Uz[l], bz[l].reshape(1, H),
                Wr[l], Ur[l], br[l].reshape(1, H),
                Wh[l], Uh[l], bh[l].reshape(1, H))
        if (l + 1) % 2 == 0:
            h, hm = _gru_ex_call(*args, mrow, mcol,
                                 W_ex[ex_i, :H], W_ex[ex_i, H:],
                                 b_ex[ex_i].reshape(1, H), wm_next)
            ex_i += 1
        else:
            h, hm = _gru_call(*args, wm_next)
    return h[:N]
